# baseline (speedup 1.0000x reference)
import functools

import jax
import jax.numpy as jnp
from jax import lax
from jax.experimental import pallas as pl
from jax.experimental.pallas import tpu as pltpu
from jax.experimental.pallas import tpu_sc as plsc

EMB_D = 64
C = 256
RB = 4
WB = 2
GAHEAD = 2


@functools.lru_cache(maxsize=None)
def _make_gather(n_mol: int, n_atom: int, V: int):
    info = plsc.get_sparse_core_info()
    NC, NS = info.num_cores, info.num_subcores
    NW = NC * NS
    n_chunks = n_mol * n_atom // C
    assert n_chunks % NW == 0 and n_mol % C == 0
    q_per_w = n_chunks // NW
    m_per_a = n_mol // C
    mesh = plsc.VectorSubcoreMesh(core_axis_name="c", subcore_axis_name="s")

    scratch = (
        [pltpu.VMEM((C,), jnp.int32) for _ in range(RB)]
        + [pltpu.VMEM((C, EMB_D), jnp.float32) for _ in range(RB)]
        + [pltpu.VMEM((EMB_D, C), jnp.float32) for _ in range(WB)]
        + [pltpu.SemaphoreType.DMA for _ in range(RB + WB)]
    )

    @functools.partial(
        pl.kernel,
        mesh=mesh,
        compiler_params=pltpu.CompilerParams(
            use_tc_tiling_on_sc=False, needs_layout_passes=False),
        out_type=jax.ShapeDtypeStruct((n_atom, EMB_D, n_mol), jnp.float32),
        scratch_types=scratch,
    )
    def k(idx_hbm, table_hbm, out_hbm, *rest):
        idx_bufs = rest[0:RB]
        row_bufs = rest[RB:2 * RB]
        col_bufs = rest[2 * RB:2 * RB + WB]
        sg = rest[2 * RB + WB:2 * RB + WB + RB]
        so = rest[2 * RB + WB + RB:]
        wid = lax.axis_index("s") * NC + lax.axis_index("c")
        q0 = wid * q_per_w

        def start_gather(j, b):
            off = (q0 + j) * C
            pltpu.sync_copy(idx_hbm.at[pl.ds(off, C)], idx_bufs[b])
            pltpu.async_copy(table_hbm.at[idx_bufs[b]], row_bufs[b], sg[b])

        def wait_gather(b):
            pltpu.make_async_copy(
                table_hbm.at[idx_bufs[b]], row_bufs[b], sg[b]).wait()

        def out_slab(j):
            q = q0 + j
            a = q // m_per_a
            m0 = (q % m_per_a) * C
            return out_hbm.at[a, :, pl.ds(m0, C)]

        def start_write(j, w):
            pltpu.async_copy(col_bufs[w], out_slab(j), so[w])

        def wait_write(j, w):
            pltpu.make_async_copy(col_bufs[w], out_slab(j), so[w]).wait()

        def transpose(b, w):
            rows = row_bufs[b]
            cols = col_bufs[w]
            lanes = lax.iota(jnp.int32, 16)

            def tbody(r0, carry):
                r_ids = r0 * 16 + lanes
                for d in range(EMB_D):
                    d_ids = jnp.full((16,), d, jnp.int32)
                    cols[d, pl.ds(r0 * 16, 16)] = plsc.load_gather(
                        rows, [r_ids, d_ids])
                return carry

            lax.fori_loop(0, C // 16, tbody, 0)

        def step(j, b, w, first, last):
            wait_gather(b)
            if not last:
                start_gather(j + GAHEAD + 1, (b + GAHEAD + 1) % RB)
            if not first:
                wait_write(j - WB, w)
            transpose(b, w)
            start_write(j, w)

        GRP = 4
        assert (q_per_w - 2 * GRP) % GRP == 0 and q_per_w >= 2 * GRP

        for j in range(GAHEAD + 1):
            start_gather(j, j % RB)
        for j in range(GRP):
            step(j, j % RB, j % WB, first=(j < WB), last=False)

        def body(gg, carry):
            jbase = GRP + gg * GRP
            for t in range(GRP):
                step(jbase + t, t % RB, t % WB, first=False, last=False)
            return carry

        lax.fori_loop(0, (q_per_w - 2 * GRP) // GRP, body, 0)

        for j in range(q_per_w - GRP, q_per_w):
            step(j, j % RB, j % WB, first=False,
                 last=(j + GAHEAD + 1 >= q_per_w))
        for j in range(q_per_w - WB, q_per_w):
            wait_write(j, j % WB)

    return k


@jax.jit
def kernel(atom_types, table):
    n_mol, n_atom = atom_types.shape
    idxT = atom_types.T.reshape(n_mol * n_atom).astype(jnp.int32)
    out3 = _make_gather(n_mol, n_atom, table.shape[0])(idxT, table)
    return jnp.transpose(out3, (2, 0, 1))


# speedup vs baseline: 2.0507x; 2.0507x over previous
import functools

import jax
import jax.numpy as jnp
from jax import lax
from jax.experimental import pallas as pl
from jax.experimental.pallas import tpu as pltpu
from jax.experimental.pallas import tpu_sc as plsc

EMB_D = 64
C = 256
RB = 4
WB = 2
GAHEAD = 2


@functools.lru_cache(maxsize=None)
def _make_gather(n_mol: int, n_atom: int, V: int):
    info = plsc.get_sparse_core_info()
    NC, NS = info.num_cores, info.num_subcores
    NW = NC * NS
    n_chunks = n_mol * n_atom // C
    assert n_chunks % NW == 0 and n_mol % C == 0
    q_per_w = n_chunks // NW
    m_per_a = n_mol // C
    mesh = plsc.VectorSubcoreMesh(core_axis_name="c", subcore_axis_name="s")

    scratch = (
        [pltpu.VMEM((C,), jnp.int32) for _ in range(RB)]
        + [pltpu.VMEM((C, EMB_D), jnp.float32) for _ in range(RB)]
        + [pltpu.VMEM((EMB_D, C), jnp.float32) for _ in range(WB)]
        + [pltpu.SemaphoreType.DMA for _ in range(RB + WB)]
    )

    @functools.partial(
        pl.kernel,
        mesh=mesh,
        compiler_params=pltpu.CompilerParams(
            use_tc_tiling_on_sc=False, needs_layout_passes=False),
        out_type=jax.ShapeDtypeStruct((n_atom, EMB_D, n_mol), jnp.float32),
        scratch_types=scratch,
    )
    def k(idx_hbm, table_hbm, out_hbm, *rest):
        idx_bufs = rest[0:RB]
        row_bufs = rest[RB:2 * RB]
        col_bufs = rest[2 * RB:2 * RB + WB]
        sg = rest[2 * RB + WB:2 * RB + WB + RB]
        so = rest[2 * RB + WB + RB:]
        wid = lax.axis_index("s") * NC + lax.axis_index("c")
        q0 = wid * q_per_w

        def start_gather(j, b):
            off = (q0 + j) * C
            pltpu.sync_copy(idx_hbm.at[pl.ds(off, C)], idx_bufs[b])
            pltpu.async_copy(table_hbm.at[idx_bufs[b]], row_bufs[b], sg[b])

        def wait_gather(b):
            pltpu.make_async_copy(
                table_hbm.at[idx_bufs[b]], row_bufs[b], sg[b]).wait()

        def out_slab(j):
            q = q0 + j
            a = q // m_per_a
            m0 = (q % m_per_a) * C
            return out_hbm.at[a, :, pl.ds(m0, C)]

        def start_write(j, w):
            pltpu.async_copy(col_bufs[w], out_slab(j), so[w])

        def wait_write(j, w):
            pltpu.make_async_copy(col_bufs[w], out_slab(j), so[w]).wait()

        lanes = lax.iota(jnp.int32, 16)
        rot = [(lanes + s) & 15 for s in range(16)]
        dvec = [d0 * 16 + lanes for d0 in range(EMB_D // 16)]

        def transpose(b, w):
            rows = row_bufs[b]
            cols = col_bufs[w]

            def tbody(r0, carry):
                m_base = r0 * 16
                for s in range(16):
                    rid = m_base + rot[s]
                    for d0 in range(EMB_D // 16):
                        v = plsc.load_gather(rows, [rid, dvec[d0]])
                        plsc.store_scatter(cols, [dvec[d0], rid], v)
                return carry

            lax.fori_loop(0, C // 16, tbody, 0)

        def step(j, b, w, first, last):
            wait_gather(b)
            if not last:
                start_gather(j + GAHEAD + 1, (b + GAHEAD + 1) % RB)
            if not first:
                wait_write(j - WB, w)
            transpose(b, w)
            start_write(j, w)

        GRP = 4
        assert (q_per_w - 2 * GRP) % GRP == 0 and q_per_w >= 2 * GRP

        for j in range(GAHEAD + 1):
            start_gather(j, j % RB)
        for j in range(GRP):
            step(j, j % RB, j % WB, first=(j < WB), last=False)

        def body(gg, carry):
            jbase = GRP + gg * GRP
            for t in range(GRP):
                step(jbase + t, t % RB, t % WB, first=False, last=False)
            return carry

        lax.fori_loop(0, (q_per_w - 2 * GRP) // GRP, body, 0)

        for j in range(q_per_w - GRP, q_per_w):
            step(j, j % RB, j % WB, first=False,
                 last=(j + GAHEAD + 1 >= q_per_w))
        for j in range(q_per_w - WB, q_per_w):
            wait_write(j, j % WB)

    return k


@jax.jit
def kernel(atom_types, table):
    n_mol, n_atom = atom_types.shape
    idxT = atom_types.T.reshape(n_mol * n_atom).astype(jnp.int32)
    out3 = _make_gather(n_mol, n_atom, table.shape[0])(idxT, table)
    return jnp.transpose(out3, (2, 0, 1))


# speedup vs baseline: 4.9733x; 2.4251x over previous
import functools

import jax
import jax.numpy as jnp
from jax import lax
from jax.experimental import pallas as pl
from jax.experimental.pallas import tpu as pltpu
from jax.experimental.pallas import tpu_sc as plsc

EMB_D = 64
C = 256
RB = 4
WB = 2
GAHEAD = 2
ND = EMB_D // 16
NDB = EMB_D // 8
MBLK = C // 128


@functools.lru_cache(maxsize=None)
def _make_gather(n_mol: int, n_atom: int, V: int):
    info = plsc.get_sparse_core_info()
    NC, NS = info.num_cores, info.num_subcores
    NW = NC * NS
    n_chunks = n_mol * n_atom // C
    assert n_chunks % NW == 0 and n_mol % C == 0
    q_per_w = n_chunks // NW
    m_per_a = n_mol // C
    a_stride = EMB_D * n_mol
    db_stride = 8 * n_mol
    mesh = plsc.VectorSubcoreMesh(core_axis_name="c", subcore_axis_name="s")

    scratch = (
        [pltpu.VMEM((C,), jnp.int32) for _ in range(RB)]
        + [pltpu.VMEM((C, EMB_D), jnp.float32) for _ in range(RB)]
        + [pltpu.VMEM((NDB, MBLK * 1024), jnp.float32) for _ in range(WB)]
        + [pltpu.SemaphoreType.DMA for _ in range(RB + WB)]
    )

    @functools.partial(
        pl.kernel,
        mesh=mesh,
        compiler_params=pltpu.CompilerParams(
            use_tc_tiling_on_sc=False, needs_layout_passes=False),
        out_type=jax.ShapeDtypeStruct((n_atom * EMB_D * n_mol,), jnp.float32),
        scratch_types=scratch,
    )
    def k(idx_hbm, table_hbm, out_hbm, *rest):
        idx_bufs = rest[0:RB]
        row_bufs = rest[RB:2 * RB]
        col_bufs = rest[2 * RB:2 * RB + WB]
        sg = rest[2 * RB + WB:2 * RB + WB + RB]
        so = rest[2 * RB + WB + RB:]
        wid = lax.axis_index("s") * NC + lax.axis_index("c")
        q0 = wid * q_per_w

        def start_gather(j, b):
            off = (q0 + j) * C
            pltpu.sync_copy(idx_hbm.at[pl.ds(off, C)], idx_bufs[b])
            pltpu.async_copy(table_hbm.at[idx_bufs[b]], row_bufs[b], sg[b])

        def wait_gather(b):
            pltpu.make_async_copy(
                table_hbm.at[idx_bufs[b]], row_bufs[b], sg[b]).wait()

        def out_runs(j):
            q = q0 + j
            a = q // m_per_a
            base = a * a_stride + (q % m_per_a) * (MBLK * 1024)
            return [base + db * db_stride for db in range(NDB)]

        def start_write(j, w):
            for db, off in enumerate(out_runs(j)):
                pltpu.async_copy(
                    col_bufs[w].at[db], out_hbm.at[pl.ds(off, MBLK * 1024)],
                    so[w])

        def wait_write(j, w):
            for db, off in enumerate(out_runs(j)):
                pltpu.make_async_copy(
                    col_bufs[w].at[db], out_hbm.at[pl.ds(off, MBLK * 1024)],
                    so[w]).wait()

        lanes = lax.iota(jnp.int32, 16)
        rot = [(lanes + s) & 15 for s in range(16)]
        dvec = [d0 * 16 + lanes for d0 in range(ND)]
        dbv = [2 * d0 + (lanes >> 3) for d0 in range(ND)]
        ibrot = [(lanes & 7) * 128 + rot[s] for s in range(16)]

        def transpose(b, w):
            rows = row_bufs[b]
            cols = col_bufs[w]

            def tbody(r0, carry):
                m_base = r0 * 16
                s_in = (r0 >> 3) * 1024 + (r0 & 7) * 16
                for s0 in range(0, 16, 4):
                    rids = [m_base + rot[s0 + i] for i in range(4)]
                    wids = [s_in + ibrot[s0 + i] for i in range(4)]
                    vs = [
                        plsc.load_gather(rows, [rids[i], dvec[d0]])
                        for i in range(4)
                        for d0 in range(ND)
                    ]
                    for i in range(4):
                        for d0 in range(ND):
                            plsc.store_scatter(
                                cols, [dbv[d0], wids[i]], vs[i * ND + d0])
                return carry

            lax.fori_loop(0, C // 16, tbody, 0)

        def step(j, b, w, first, last):
            wait_gather(b)
            if not last:
                start_gather(j + GAHEAD + 1, (b + GAHEAD + 1) % RB)
            if not first:
                wait_write(j - WB, w)
            transpose(b, w)
            start_write(j, w)

        GRP = 4
        assert (q_per_w - 2 * GRP) % GRP == 0 and q_per_w >= 2 * GRP

        for j in range(GAHEAD + 1):
            start_gather(j, j % RB)
        for j in range(GRP):
            step(j, j % RB, j % WB, first=(j < WB), last=False)

        def body(gg, carry):
            jbase = GRP + gg * GRP
            for t in range(GRP):
                step(jbase + t, t % RB, t % WB, first=False, last=False)
            return carry

        lax.fori_loop(0, (q_per_w - 2 * GRP) // GRP, body, 0)

        for j in range(q_per_w - GRP, q_per_w):
            step(j, j % RB, j % WB, first=False,
                 last=(j + GAHEAD + 1 >= q_per_w))
        for j in range(q_per_w - WB, q_per_w):
            wait_write(j, j % WB)

    return k


@jax.jit
def kernel(atom_types, table):
    n_mol, n_atom = atom_types.shape
    idxT = atom_types.T.reshape(n_mol * n_atom).astype(jnp.int32)
    flat = _make_gather(n_mol, n_atom, table.shape[0])(idxT, table)
    out5 = flat.reshape(n_atom, EMB_D // 8, n_mol // 128, 8, 128)
    return jnp.transpose(out5, (2, 4, 0, 1, 3)).reshape(n_mol, n_atom, EMB_D)


# speedup vs baseline: 4.9782x; 1.0010x over previous
import functools

import jax
import jax.numpy as jnp
from jax import lax
from jax.experimental import pallas as pl
from jax.experimental.pallas import tpu as pltpu
from jax.experimental.pallas import tpu_sc as plsc

EMB_D = 64
C = 256
RB = 4
WB = 2
GAHEAD = 2
ND = EMB_D // 16
NDB = EMB_D // 8
MBLK = C // 128


@functools.lru_cache(maxsize=None)
def _make_gather(n_mol: int, n_atom: int, V: int):
    info = plsc.get_sparse_core_info()
    NC, NS = info.num_cores, info.num_subcores
    NW = NC * NS
    n_chunks = n_mol * n_atom // C
    assert n_chunks % NW == 0 and n_mol % C == 0
    q_per_w = n_chunks // NW
    m_per_a = n_mol // C
    a_stride = EMB_D * n_mol
    db_stride = 8 * n_mol
    mesh = plsc.VectorSubcoreMesh(core_axis_name="c", subcore_axis_name="s")

    scratch = (
        [pltpu.VMEM((C,), jnp.int32) for _ in range(RB)]
        + [pltpu.VMEM((C, EMB_D), jnp.float32) for _ in range(RB)]
        + [pltpu.VMEM((NDB * MBLK * 1024,), jnp.float32) for _ in range(WB)]
        + [pltpu.SemaphoreType.DMA for _ in range(RB + WB)]
    )

    @functools.partial(
        pl.kernel,
        mesh=mesh,
        compiler_params=pltpu.CompilerParams(
            use_tc_tiling_on_sc=False, needs_layout_passes=False),
        out_type=jax.ShapeDtypeStruct((n_atom * EMB_D * n_mol,), jnp.float32),
        scratch_types=scratch,
    )
    def k(idx_hbm, table_hbm, out_hbm, *rest):
        idx_bufs = rest[0:RB]
        row_bufs = rest[RB:2 * RB]
        col_bufs = rest[2 * RB:2 * RB + WB]
        sg = rest[2 * RB + WB:2 * RB + WB + RB]
        so = rest[2 * RB + WB + RB:]
        wid = lax.axis_index("s") * NC + lax.axis_index("c")
        q0 = wid * q_per_w

        def start_gather(j, b):
            off = (q0 + j) * C
            pltpu.sync_copy(idx_hbm.at[pl.ds(off, C)], idx_bufs[b])
            pltpu.async_copy(table_hbm.at[idx_bufs[b]], row_bufs[b], sg[b])

        def wait_gather(b):
            pltpu.make_async_copy(
                table_hbm.at[idx_bufs[b]], row_bufs[b], sg[b]).wait()

        def out_runs(j):
            q = q0 + j
            a = q // m_per_a
            base = a * a_stride + (q % m_per_a) * (MBLK * 1024)
            return [base + db * db_stride for db in range(NDB)]

        def start_write(j, w):
            for db, off in enumerate(out_runs(j)):
                pltpu.async_copy(
                    col_bufs[w].at[pl.ds(db * MBLK * 1024, MBLK * 1024)],
                    out_hbm.at[pl.ds(off, MBLK * 1024)], so[w])

        def wait_write(j, w):
            for db, off in enumerate(out_runs(j)):
                pltpu.make_async_copy(
                    col_bufs[w].at[pl.ds(db * MBLK * 1024, MBLK * 1024)],
                    out_hbm.at[pl.ds(off, MBLK * 1024)], so[w]).wait()

        lanes = lax.iota(jnp.int32, 16)
        rot = [(lanes + s) & 15 for s in range(16)]
        dvec = [d0 * 16 + lanes for d0 in range(ND)]
        wrot = [
            (lanes >> 3) * (MBLK * 1024) + (lanes & 7) * 128 + rot[s]
            for s in range(16)
        ]
        wdflat = [2 * d0 * (MBLK * 1024) for d0 in range(ND)]

        def transpose(b, w):
            rows = row_bufs[b]
            cols = col_bufs[w]

            def tbody(r0, carry):
                m_base = r0 * 16
                s_in = (r0 >> 3) * 1024 + (r0 & 7) * 16
                for s0 in range(0, 16, 4):
                    rids = [m_base + rot[s0 + i] for i in range(4)]
                    wids = [s_in + wrot[s0 + i] for i in range(4)]
                    vs = [
                        plsc.load_gather(rows, [rids[i], dvec[d0]])
                        for i in range(4)
                        for d0 in range(ND)
                    ]
                    for i in range(4):
                        for d0 in range(ND):
                            plsc.store_scatter(
                                cols, [wids[i] + wdflat[d0]], vs[i * ND + d0])
                return carry

            lax.fori_loop(0, C // 16, tbody, 0)

        def step(j, b, w, first, last):
            wait_gather(b)
            if not last:
                start_gather(j + GAHEAD + 1, (b + GAHEAD + 1) % RB)
            if not first:
                wait_write(j - WB, w)
            transpose(b, w)
            start_write(j, w)

        GRP = 4
        assert (q_per_w - 2 * GRP) % GRP == 0 and q_per_w >= 2 * GRP

        for j in range(GAHEAD + 1):
            start_gather(j, j % RB)
        for j in range(GRP):
            step(j, j % RB, j % WB, first=(j < WB), last=False)

        def body(gg, carry):
            jbase = GRP + gg * GRP
            for t in range(GRP):
                step(jbase + t, t % RB, t % WB, first=False, last=False)
            return carry

        lax.fori_loop(0, (q_per_w - 2 * GRP) // GRP, body, 0)

        for j in range(q_per_w - GRP, q_per_w):
            step(j, j % RB, j % WB, first=False,
                 last=(j + GAHEAD + 1 >= q_per_w))
        for j in range(q_per_w - WB, q_per_w):
            wait_write(j, j % WB)

    return k


@jax.jit
def kernel(atom_types, table):
    n_mol, n_atom = atom_types.shape
    idxT = atom_types.T.reshape(n_mol * n_atom).astype(jnp.int32)
    flat = _make_gather(n_mol, n_atom, table.shape[0])(idxT, table)
    out5 = flat.reshape(n_atom, EMB_D // 8, n_mol // 128, 8, 128)
    return jnp.transpose(out5, (2, 4, 0, 1, 3)).reshape(n_mol, n_atom, EMB_D)


# speedup vs baseline: 5.0225x; 1.0089x over previous
import functools

import jax
import jax.numpy as jnp
from jax import lax
from jax.experimental import pallas as pl
from jax.experimental.pallas import tpu as pltpu
from jax.experimental.pallas import tpu_sc as plsc

EMB_D = 64
C = 256
RB = 5
WB = 2
GAHEAD = 3
ND = EMB_D // 16
NDB = EMB_D // 8
MBLK = C // 128


@functools.lru_cache(maxsize=None)
def _make_gather(n_mol: int, n_atom: int, V: int):
    info = plsc.get_sparse_core_info()
    NC, NS = info.num_cores, info.num_subcores
    NW = NC * NS
    n_chunks = n_mol * n_atom // C
    assert n_chunks % NW == 0 and n_mol % C == 0
    q_per_w = n_chunks // NW
    m_per_a = n_mol // C
    a_stride = EMB_D * n_mol
    db_stride = 8 * n_mol
    mesh = plsc.VectorSubcoreMesh(core_axis_name="c", subcore_axis_name="s")

    scratch = (
        [pltpu.VMEM((C,), jnp.int32) for _ in range(RB)]
        + [pltpu.VMEM((C, EMB_D), jnp.float32) for _ in range(RB)]
        + [pltpu.VMEM((NDB * MBLK * 1024,), jnp.float32) for _ in range(WB)]
        + [pltpu.SemaphoreType.DMA for _ in range(RB + WB)]
    )

    @functools.partial(
        pl.kernel,
        mesh=mesh,
        compiler_params=pltpu.CompilerParams(
            use_tc_tiling_on_sc=False, needs_layout_passes=False),
        out_type=jax.ShapeDtypeStruct((n_atom * EMB_D * n_mol,), jnp.float32),
        scratch_types=scratch,
    )
    def k(idx_hbm, table_hbm, out_hbm, *rest):
        idx_bufs = rest[0:RB]
        row_bufs = rest[RB:2 * RB]
        col_bufs = rest[2 * RB:2 * RB + WB]
        sg = rest[2 * RB + WB:2 * RB + WB + RB]
        so = rest[2 * RB + WB + RB:]
        wid = lax.axis_index("s") * NC + lax.axis_index("c")
        q0 = wid * q_per_w

        def start_gather(j, b):
            off = (q0 + j) * C
            pltpu.sync_copy(idx_hbm.at[pl.ds(off, C)], idx_bufs[b])
            pltpu.async_copy(table_hbm.at[idx_bufs[b]], row_bufs[b], sg[b])

        def wait_gather(b):
            pltpu.make_async_copy(
                table_hbm.at[idx_bufs[b]], row_bufs[b], sg[b]).wait()

        def out_runs(j):
            q = q0 + j
            a = q // m_per_a
            base = a * a_stride + (q % m_per_a) * (MBLK * 1024)
            return [base + db * db_stride for db in range(NDB)]

        def start_write(j, w):
            for db, off in enumerate(out_runs(j)):
                pltpu.async_copy(
                    col_bufs[w].at[pl.ds(db * MBLK * 1024, MBLK * 1024)],
                    out_hbm.at[pl.ds(off, MBLK * 1024)], so[w])

        def wait_write(j, w):
            for db, off in enumerate(out_runs(j)):
                pltpu.make_async_copy(
                    col_bufs[w].at[pl.ds(db * MBLK * 1024, MBLK * 1024)],
                    out_hbm.at[pl.ds(off, MBLK * 1024)], so[w]).wait()

        lanes = lax.iota(jnp.int32, 16)
        rot = [(lanes + s) & 15 for s in range(16)]
        dvec = [d0 * 16 + lanes for d0 in range(ND)]
        wrot = [
            (lanes >> 3) * (MBLK * 1024) + (lanes & 7) * 128 + rot[s]
            for s in range(16)
        ]
        wdflat = [2 * d0 * (MBLK * 1024) for d0 in range(ND)]

        def transpose(b, w):
            rows = row_bufs[b]
            cols = col_bufs[w]

            def tbody(r0, carry):
                m_base = r0 * 16
                s_in = (r0 >> 3) * 1024 + (r0 & 7) * 16
                for s0 in range(0, 16, 4):
                    rids = [m_base + rot[s0 + i] for i in range(4)]
                    wids = [s_in + wrot[s0 + i] for i in range(4)]
                    vs = [
                        plsc.load_gather(rows, [rids[i], dvec[d0]])
                        for i in range(4)
                        for d0 in range(ND)
                    ]
                    for i in range(4):
                        for d0 in range(ND):
                            plsc.store_scatter(
                                cols, [wids[i] + wdflat[d0]], vs[i * ND + d0])
                return carry

            lax.fori_loop(0, C // 16, tbody, 0)

        def step(j, b, w, first, last):
            wait_gather(b)
            if not last:
                start_gather(j + GAHEAD + 1, (b + GAHEAD + 1) % RB)
            if not first:
                wait_write(j - WB, w)
            transpose(b, w)
            start_write(j, w)

        GRP = 10
        assert (q_per_w - 2 * GRP) % GRP == 0 and q_per_w >= 2 * GRP

        for j in range(GAHEAD + 1):
            start_gather(j, j % RB)
        for j in range(GRP):
            step(j, j % RB, j % WB, first=(j < WB), last=False)

        def body(gg, carry):
            jbase = GRP + gg * GRP
            for t in range(GRP):
                step(jbase + t, t % RB, t % WB, first=False, last=False)
            return carry

        lax.fori_loop(0, (q_per_w - 2 * GRP) // GRP, body, 0)

        for j in range(q_per_w - GRP, q_per_w):
            step(j, j % RB, j % WB, first=False,
                 last=(j + GAHEAD + 1 >= q_per_w))
        for j in range(q_per_w - WB, q_per_w):
            wait_write(j, j % WB)

    return k


@jax.jit
def kernel(atom_types, table):
    n_mol, n_atom = atom_types.shape
    idxT = atom_types.T.reshape(n_mol * n_atom).astype(jnp.int32)
    flat = _make_gather(n_mol, n_atom, table.shape[0])(idxT, table)
    out5 = flat.reshape(n_atom, EMB_D // 8, n_mol // 128, 8, 128)
    return jnp.transpose(out5, (2, 4, 0, 1, 3)).reshape(n_mol, n_atom, EMB_D)


# speedup vs baseline: 5.9936x; 1.1934x over previous
import functools

import jax
import jax.numpy as jnp
from jax import lax
from jax.experimental import pallas as pl
from jax.experimental.pallas import tpu as pltpu
from jax.experimental.pallas import tpu_sc as plsc

EMB_D = 64
C = 256
RB = 5
WB = 2
GAHEAD = 3
ND = EMB_D // 16
NDB = EMB_D // 8
MBLK = C // 128


@functools.lru_cache(maxsize=None)
def _make_gather(n_mol: int, n_atom: int, V: int):
    info = plsc.get_sparse_core_info()
    NC, NS = info.num_cores, info.num_subcores
    NW = NC * NS
    n_chunks = n_mol * n_atom // C
    assert n_chunks % NW == 0 and n_mol % C == 0
    q_per_w = n_chunks // NW
    m_per_a = n_mol // C
    a_stride = EMB_D * n_mol
    db_stride = 8 * n_mol
    mesh = plsc.VectorSubcoreMesh(core_axis_name="c", subcore_axis_name="s")

    scratch = (
        [pltpu.VMEM((C,), jnp.int32) for _ in range(RB)]
        + [pltpu.VMEM((C, EMB_D), jnp.float32) for _ in range(RB)]
        + [pltpu.VMEM((NDB * MBLK * 1024,), jnp.float32) for _ in range(WB)]
        + [pltpu.SemaphoreType.DMA for _ in range(2 * RB + WB)]
    )

    @functools.partial(
        pl.kernel,
        mesh=mesh,
        compiler_params=pltpu.CompilerParams(
            use_tc_tiling_on_sc=False, needs_layout_passes=False),
        out_type=jax.ShapeDtypeStruct((n_atom * EMB_D * n_mol,), jnp.float32),
        scratch_types=scratch,
    )
    def k(idx_hbm, table_hbm, out_hbm, *rest):
        idx_bufs = rest[0:RB]
        row_bufs = rest[RB:2 * RB]
        col_bufs = rest[2 * RB:2 * RB + WB]
        sg = rest[2 * RB + WB:3 * RB + WB]
        si = rest[3 * RB + WB:4 * RB + WB]
        so = rest[4 * RB + WB:]
        wid = lax.axis_index("s") * NC + lax.axis_index("c")
        q0 = wid * q_per_w

        def stage_idx(j, b):
            off = (q0 + j) * C
            pltpu.async_copy(idx_hbm.at[pl.ds(off, C)], idx_bufs[b], si[b])

        def start_gather(j, b):
            off = (q0 + j) * C
            pltpu.make_async_copy(
                idx_hbm.at[pl.ds(off, C)], idx_bufs[b], si[b]).wait()
            pltpu.async_copy(table_hbm.at[idx_bufs[b]], row_bufs[b], sg[b])

        def wait_gather(b):
            pltpu.make_async_copy(
                table_hbm.at[idx_bufs[b]], row_bufs[b], sg[b]).wait()

        def out_runs(j):
            q = q0 + j
            a = q // m_per_a
            base = a * a_stride + (q % m_per_a) * (MBLK * 1024)
            return [base + db * db_stride for db in range(NDB)]

        def start_write(j, w):
            for db, off in enumerate(out_runs(j)):
                pltpu.async_copy(
                    col_bufs[w].at[pl.ds(db * MBLK * 1024, MBLK * 1024)],
                    out_hbm.at[pl.ds(off, MBLK * 1024)], so[w])

        def wait_write(j, w):
            for db, off in enumerate(out_runs(j)):
                pltpu.make_async_copy(
                    col_bufs[w].at[pl.ds(db * MBLK * 1024, MBLK * 1024)],
                    out_hbm.at[pl.ds(off, MBLK * 1024)], so[w]).wait()

        lanes = lax.iota(jnp.int32, 16)
        rot = [(lanes + s) & 15 for s in range(16)]
        dvec = [d0 * 16 + lanes for d0 in range(ND)]
        wrot = [
            (lanes >> 3) * (MBLK * 1024) + (lanes & 7) * 128 + rot[s]
            for s in range(16)
        ]
        wdflat = [2 * d0 * (MBLK * 1024) for d0 in range(ND)]

        def transpose(b, w):
            rows = row_bufs[b]
            cols = col_bufs[w]

            def tbody(r0, carry):
                m_base = r0 * 16
                s_in = (r0 >> 3) * 1024 + (r0 & 7) * 16
                for s0 in range(0, 16, 4):
                    rids = [m_base + rot[s0 + i] for i in range(4)]
                    wids = [s_in + wrot[s0 + i] for i in range(4)]
                    vs = [
                        plsc.load_gather(rows, [rids[i], dvec[d0]])
                        for i in range(4)
                        for d0 in range(ND)
                    ]
                    for i in range(4):
                        for d0 in range(ND):
                            plsc.store_scatter(
                                cols, [wids[i] + wdflat[d0]], vs[i * ND + d0])
                return carry

            lax.fori_loop(0, C // 16, tbody, 0)

        def step(j, b, w, first, last, last2):
            wait_gather(b)
            if not last:
                start_gather(j + GAHEAD + 1, (b + GAHEAD + 1) % RB)
            if not last2:
                stage_idx(j + GAHEAD + 2, (b + GAHEAD + 2) % RB)
            if not first:
                wait_write(j - WB, w)
            transpose(b, w)
            start_write(j, w)

        GRP = 10
        assert (q_per_w - 2 * GRP) % GRP == 0 and q_per_w >= 2 * GRP

        for j in range(GAHEAD + 2):
            stage_idx(j, j % RB)
        for j in range(GAHEAD + 1):
            start_gather(j, j % RB)
        for j in range(GRP):
            step(j, j % RB, j % WB, first=(j < WB), last=False, last2=False)

        def body(gg, carry):
            jbase = GRP + gg * GRP
            for t in range(GRP):
                step(jbase + t, t % RB, t % WB,
                     first=False, last=False, last2=False)
            return carry

        lax.fori_loop(0, (q_per_w - 2 * GRP) // GRP, body, 0)

        for j in range(q_per_w - GRP, q_per_w):
            step(j, j % RB, j % WB, first=False,
                 last=(j + GAHEAD + 1 >= q_per_w),
                 last2=(j + GAHEAD + 2 >= q_per_w))
        for j in range(q_per_w - WB, q_per_w):
            wait_write(j, j % WB)

    return k


@jax.jit
def kernel(atom_types, table):
    n_mol, n_atom = atom_types.shape
    idxT = atom_types.T.reshape(n_mol * n_atom).astype(jnp.int32)
    flat = _make_gather(n_mol, n_atom, table.shape[0])(idxT, table)
    out5 = flat.reshape(n_atom, EMB_D // 8, n_mol // 128, 8, 128)
    return jnp.transpose(out5, (2, 4, 0, 1, 3)).reshape(n_mol, n_atom, EMB_D)
